# baseline (speedup 1.0000x reference)
import functools

import jax
import jax.numpy as jnp
from jax import lax
from jax.experimental import pallas as pl
from jax.experimental.pallas import tpu as pltpu
from jax.experimental.pallas import tpu_sc as plsc

_V = 1000000
_D = 64
_DP = 128
_H = 50
_HP = 56
_BATCH = 16384
_NW = 32
_BPW = _BATCH // _NW
_CB = 4
_NCHK = _BPW // _CB
_NBUF = 2
_RPW = _BPW * _HP
_RPC = _CB * _HP
_PAD_BLK = 4000
_SLC_BLK = 64


def _pad_tc(table):
    def body(t_ref, o_ref):
        o_ref[:, : _D] = t_ref[...]

    return pl.pallas_call(
        body,
        grid=(_V // _PAD_BLK,),
        in_specs=[pl.BlockSpec((_PAD_BLK, _D), lambda i: (i, 0))],
        out_specs=pl.BlockSpec((_PAD_BLK, _DP), lambda i: (i, 0)),
        out_shape=jax.ShapeDtypeStruct((_V, _DP), jnp.float32),
    )(table)


def _slice_tc(out3):
    def body(i_ref, o_ref):
        o_ref[...] = i_ref[:, : _H, : _D]

    return pl.pallas_call(
        body,
        grid=(_BATCH // _SLC_BLK,),
        in_specs=[pl.BlockSpec((_SLC_BLK, _HP, _DP), lambda i: (i, 0, 0))],
        out_specs=pl.BlockSpec((_SLC_BLK, _H, _D), lambda i: (i, 0, 0)),
        out_shape=jax.ShapeDtypeStruct((_BATCH, _H, _D), jnp.float32),
    )(out3)


def _gather_sc(idx_grp, table_p):
    mesh = plsc.VectorSubcoreMesh(core_axis_name="c", subcore_axis_name="s")

    @functools.partial(
        pl.kernel,
        mesh=mesh,
        out_type=jax.ShapeDtypeStruct((_BATCH * _HP, _DP), jnp.float32),
        compiler_params=pltpu.CompilerParams(use_tc_tiling_on_sc=False),
        scratch_types=[
            pltpu.VMEM((_RPW,), jnp.int32),
            pltpu.VMEM((_NBUF, _RPC, _DP), jnp.float32),
            pltpu.SemaphoreType.DMA,
            pltpu.SemaphoreType.DMA,
        ],
    )
    def k(idx_hbm, table_hbm, out_hbm, idx_v, rows_v, gsem0, gsem1):
        gsems = (gsem0, gsem1)
        wid = lax.axis_index("s") * 2 + lax.axis_index("c")
        base_r = wid * _RPW
        pltpu.sync_copy(idx_hbm.at[wid], idx_v)

        def start_gather(g, nb):
            for j in range(2):
                pltpu.async_copy(
                    table_hbm.at[idx_v.at[pl.ds(g * _RPC + j * 112, 112)]],
                    rows_v.at[nb].at[pl.ds(j * 112, 112)],
                    gsems[nb],
                )

        def wait_gather(g, nb):
            for j in range(2):
                pltpu.make_async_copy(
                    table_hbm.at[idx_v.at[pl.ds(g * _RPC + j * 112, 112)]],
                    rows_v.at[nb].at[pl.ds(j * 112, 112)],
                    gsems[nb],
                ).wait()

        for nb in range(_NBUF):
            start_gather(nb, nb)

        def body(t, carry):
            for nb in range(_NBUF):
                g = t * _NBUF + nb
                wait_gather(g, nb)
                pltpu.sync_copy(
                    rows_v.at[nb],
                    out_hbm.at[pl.ds(base_r + g * _RPC, _RPC)],
                )

                @pl.when(g + _NBUF < _NCHK)
                def _():
                    start_gather(g + _NBUF, nb)

            return carry

        lax.fori_loop(0, _NCHK // _NBUF, body, 0)

    return k(idx_grp, table_p)


def kernel(input, table):
    idx56 = jnp.pad(input.astype(jnp.int32), ((0, 0), (0, _HP - _H)))
    idx_grp = idx56.reshape(_NW, _RPW)
    table_p = _pad_tc(table)
    out2 = _gather_sc(idx_grp, table_p)
    out3 = out2.reshape(_BATCH, _HP, _DP)
    return _slice_tc(out3)


# speedup vs baseline: 4.7995x; 4.7995x over previous
import functools

import jax
import jax.numpy as jnp
from jax import lax
from jax.experimental import pallas as pl
from jax.experimental.pallas import tpu as pltpu
from jax.experimental.pallas import tpu_sc as plsc

_D = 64
_NW = 32
_B = 16384 * 50
_BPW = _B // _NW
_SUB = 128
_SPC = 4
_CH = _SUB * _SPC
_NCH = _BPW // _CH
_NBUF = 2


def _gather_sc(idx_grp, table):
    mesh = plsc.VectorSubcoreMesh(core_axis_name="c", subcore_axis_name="s")

    @functools.partial(
        pl.kernel,
        mesh=mesh,
        out_type=jax.ShapeDtypeStruct((_B, _D), jnp.float32),
        compiler_params=pltpu.CompilerParams(
            use_tc_tiling_on_sc=False,
            skip_device_barrier=True,
            disable_bounds_checks=True,
            disable_semaphore_checks=True,
        ),
        scratch_types=[
            pltpu.VMEM((_NCH * _SPC, _SUB), jnp.int32),
            pltpu.VMEM((_NBUF, _CH, _D), jnp.float32),
            pltpu.SemaphoreType.DMA,
            pltpu.SemaphoreType.DMA,
        ],
    )
    def k(idx_hbm, table_hbm, out_hbm, idx_v, rows_v, gsem0, gsem1):
        gsems = (gsem0, gsem1)
        wid = lax.axis_index("s") * 2 + lax.axis_index("c")
        base = wid * _BPW
        pltpu.sync_copy(idx_hbm.at[wid], idx_v)

        def start_gather(g, b):
            for j in range(_SPC):
                pltpu.async_copy(
                    table_hbm.at[idx_v.at[g * _SPC + j]],
                    rows_v.at[b].at[pl.ds(j * _SUB, _SUB)],
                    gsems[b],
                )

        def wait_gather(g, b):
            for j in range(_SPC):
                pltpu.make_async_copy(
                    table_hbm.at[idx_v.at[g * _SPC + j]],
                    rows_v.at[b].at[pl.ds(j * _SUB, _SUB)],
                    gsems[b],
                ).wait()

        for b in range(_NBUF):
            start_gather(b, b)

        def body(t, carry):
            for b in range(_NBUF):
                g = t * _NBUF + b
                wait_gather(g, b)
                pltpu.sync_copy(
                    rows_v.at[b], out_hbm.at[pl.ds(base + g * _CH, _CH)]
                )

                @pl.when(g + _NBUF < _NCH)
                def _():
                    start_gather(g + _NBUF, b)

            return carry

        lax.fori_loop(0, _NCH // _NBUF, body, 0)

    return k(idx_grp, table)


def kernel(input, table):
    idx = input.reshape(-1).astype(jnp.int32).reshape(_NW, _NCH * _SPC, _SUB)
    out = _gather_sc(idx, table)
    return out.reshape(input.shape[0], input.shape[1], _D)
